# Initial kernel scaffold; baseline (speedup 1.0000x reference)
import functools
import math

import jax
import jax.numpy as jnp
from jax import lax
from jax.experimental import pallas as pl
from jax.experimental.pallas import tpu as pltpu
from jax.experimental.pallas import tpu_sc as plsc

BACKOFF_PROB = 1e-10
CLAMP = 20.0

_NC = 2
_NS = 16
_NW = _NC * _NS


def _sc_gather_body(idx_hbm, w_hbm, b_hbm, n_hbm,
                    w_out, b_out, n_out,
                    idx_v, rows_v, b_v, n_v, sem0, sem1, sem2, b_per_w):
    wid = lax.axis_index("s") * _NC + lax.axis_index("c")
    base = wid * b_per_w
    pltpu.sync_copy(idx_hbm.at[pl.ds(base, b_per_w)], idx_v)
    cp0 = pltpu.async_copy(w_hbm.at[idx_v], rows_v, sem0)
    cp1 = pltpu.async_copy(b_hbm.at[idx_v], b_v, sem1)
    cp2 = pltpu.async_copy(n_hbm.at[idx_v], n_v, sem2)
    cp0.wait()
    cp1.wait()
    cp2.wait()
    pltpu.sync_copy(rows_v, w_out.at[pl.ds(base, b_per_w)])
    pltpu.sync_copy(b_v, b_out.at[pl.ds(base, b_per_w)])
    pltpu.sync_copy(n_v, n_out.at[pl.ds(base, b_per_w)])


def _make_sc_gather(total, d):
    b_per_w = total // _NW
    mesh = plsc.VectorSubcoreMesh(core_axis_name="c", subcore_axis_name="s")
    return pl.kernel(
        functools.partial(_sc_gather_body, b_per_w=b_per_w),
        mesh=mesh,
        out_type=[
            jax.ShapeDtypeStruct((total, d), jnp.float32),
            jax.ShapeDtypeStruct((total,), jnp.float32),
            jax.ShapeDtypeStruct((total,), jnp.float32),
        ],
        scratch_types=[
            pltpu.VMEM((b_per_w,), jnp.int32),
            pltpu.VMEM((b_per_w, d), jnp.float32),
            pltpu.VMEM((b_per_w,), jnp.float32),
            pltpu.VMEM((b_per_w,), jnp.float32),
            pltpu.SemaphoreType.DMA,
            pltpu.SemaphoreType.DMA,
            pltpu.SemaphoreType.DMA,
        ],
    )


def _tc_loss_body(x_ref, w_ref, bt_ref, nt_ref, bn_ref, nn_ref, out_ref,
                  *, bn_count, k, kpad, norm_term):
    x = x_ref[...]
    wt = w_ref[0:bn_count, :]
    wn = w_ref[bn_count:bn_count + kpad, :]
    ts = jnp.sum(x * wt, axis=1, keepdims=True) + bt_ref[...]
    ns = lax.dot_general(x, wn, (((1,), (1,)), ((), ())),
                         precision=lax.Precision.HIGHEST,
                         preferred_element_type=jnp.float32)
    ns = ns + bn_ref[...]
    pm = jnp.exp(jnp.minimum(ts - norm_term, CLAMP))
    pnm = jnp.exp(jnp.minimum(ns - norm_term, CLAMP))
    kpt = k * nt_ref[...]
    kpn = k * nn_ref[...]
    p_true = pm / (pm + kpt + BACKOFF_PROB)
    p_noise = kpn / (pnm + kpn + BACKOFF_PROB)
    lp = jnp.log(p_noise + BACKOFF_PROB)
    col = lax.broadcasted_iota(jnp.int32, lp.shape, 1)
    lp = jnp.where(col < k, lp, 0.0)
    loss = -(jnp.log(p_true + BACKOFF_PROB) + jnp.sum(lp, axis=1, keepdims=True))
    out_ref[0, 0] = jnp.sum(loss) / bn_count


def kernel(target, input, weight, bias, noise, noise_idx):
    b, n, d = input.shape
    bn_count = b * n
    v = weight.shape[0]
    k = noise_idx.shape[0]
    norm_term = float(math.log(v))

    total = ((bn_count + k + 8 * _NW - 1) // (8 * _NW)) * (8 * _NW)
    kpad = total - bn_count

    tflat = target.reshape(-1).astype(jnp.int32)
    nidx = jnp.pad(noise_idx.astype(jnp.int32), (0, kpad - k))
    idx_all = jnp.concatenate([tflat, nidx])

    w_rows, b_rows, n_rows = _make_sc_gather(total, d)(idx_all, weight, bias, noise)

    x2 = input.reshape(bn_count, d)
    bt = b_rows[:bn_count].reshape(bn_count, 1)
    nt = n_rows[:bn_count].reshape(bn_count, 1)
    bn_vals = b_rows[bn_count:].reshape(1, kpad)
    nn_vals = n_rows[bn_count:].reshape(1, kpad)

    out = pl.pallas_call(
        functools.partial(_tc_loss_body, bn_count=bn_count, k=k, kpad=kpad,
                          norm_term=norm_term),
        out_shape=jax.ShapeDtypeStruct((1, 1), jnp.float32),
    )(x2, w_rows, bt, nt, bn_vals, nn_vals)
    return out[0, 0]


# speedup vs baseline: 1.1191x; 1.1191x over previous
import functools
import math

import jax
import jax.numpy as jnp
from jax import lax
from jax.experimental import pallas as pl
from jax.experimental.pallas import tpu as pltpu
from jax.experimental.pallas import tpu_sc as plsc

BACKOFF_PROB = 1e-10
CLAMP = 20.0

_NC = 2
_NS = 16
_NW = _NC * _NS


def _sc_gather_body(idx_hbm, w_hbm, b_hbm, n_hbm,
                    w_out, b_out, n_out,
                    idx_v, rows_v, b_v, n_v, sem0, sem1, sem2, b_per_w):
    wid = lax.axis_index("s") * _NC + lax.axis_index("c")
    base = wid * b_per_w
    pltpu.sync_copy(idx_hbm.at[pl.ds(base, b_per_w)], idx_v)
    cp0 = pltpu.async_copy(w_hbm.at[idx_v], rows_v, sem0)
    cp1 = pltpu.async_copy(b_hbm.at[idx_v], b_v, sem1)
    cp2 = pltpu.async_copy(n_hbm.at[idx_v], n_v, sem2)
    cp0.wait()
    cp1.wait()
    cp2.wait()
    pltpu.sync_copy(rows_v, w_out.at[pl.ds(base, b_per_w)])
    pltpu.sync_copy(b_v, b_out.at[pl.ds(base, b_per_w)])
    pltpu.sync_copy(n_v, n_out.at[pl.ds(base, b_per_w)])


def _make_sc_gather(total, d):
    b_per_w = total // _NW
    mesh = plsc.VectorSubcoreMesh(core_axis_name="c", subcore_axis_name="s")
    return pl.kernel(
        functools.partial(_sc_gather_body, b_per_w=b_per_w),
        mesh=mesh,
        out_type=[
            jax.ShapeDtypeStruct((total, d), jnp.float32),
            jax.ShapeDtypeStruct((total,), jnp.float32),
            jax.ShapeDtypeStruct((total,), jnp.float32),
        ],
        scratch_types=[
            pltpu.VMEM((b_per_w,), jnp.int32),
            pltpu.VMEM((b_per_w, d), jnp.float32),
            pltpu.VMEM((b_per_w,), jnp.float32),
            pltpu.VMEM((b_per_w,), jnp.float32),
            pltpu.SemaphoreType.DMA,
            pltpu.SemaphoreType.DMA,
            pltpu.SemaphoreType.DMA,
        ],
    )


def _tc_loss_body(x_ref, w_ref, bt_ref, nt_ref, bn_ref, nn_ref, out_ref,
                  *, bn_count, k, kpad, norm_term):
    x = x_ref[...]
    wt = w_ref[0:bn_count, :]
    wn = w_ref[bn_count:bn_count + kpad, :]
    ts = jnp.sum(x * wt, axis=1, keepdims=True) + bt_ref[...]
    ns = lax.dot_general(x, wn, (((1,), (1,)), ((), ())),
                         precision=lax.Precision.HIGHEST,
                         preferred_element_type=jnp.float32)
    ns = ns + bn_ref[...]
    pm = jnp.exp(jnp.minimum(ts - norm_term, CLAMP))
    pnm = jnp.exp(jnp.minimum(ns - norm_term, CLAMP))
    kpt = k * nt_ref[...]
    kpn = k * nn_ref[...]
    p_true = pm / (pm + kpt + BACKOFF_PROB)
    p_noise = kpn / (pnm + kpn + BACKOFF_PROB)
    lp = jnp.log(p_noise + BACKOFF_PROB)
    col = lax.broadcasted_iota(jnp.int32, lp.shape, 1)
    lp = jnp.where(col < k, lp, 0.0)
    loss = -(jnp.log(p_true + BACKOFF_PROB) + jnp.sum(lp, axis=1, keepdims=True))
    out_ref[...] = jnp.sum(loss, axis=(0, 1), keepdims=True) / bn_count


def kernel(target, input, weight, bias, noise, noise_idx):
    b, n, d = input.shape
    bn_count = b * n
    v = weight.shape[0]
    k = noise_idx.shape[0]
    norm_term = float(math.log(v))

    total = ((bn_count + k + 8 * _NW - 1) // (8 * _NW)) * (8 * _NW)
    kpad = total - bn_count

    tflat = target.reshape(-1).astype(jnp.int32)
    nidx = jnp.pad(noise_idx.astype(jnp.int32), (0, kpad - k))
    idx_all = jnp.concatenate([tflat, nidx])

    w_rows, b_rows, n_rows = _make_sc_gather(total, d)(idx_all, weight, bias, noise)

    x2 = input.reshape(bn_count, d)
    bt = b_rows[:bn_count].reshape(bn_count, 1)
    nt = n_rows[:bn_count].reshape(bn_count, 1)
    bn_vals = b_rows[bn_count:].reshape(1, kpad)
    nn_vals = n_rows[bn_count:].reshape(1, kpad)

    out = pl.pallas_call(
        functools.partial(_tc_loss_body, bn_count=bn_count, k=k, kpad=kpad,
                          norm_term=norm_term),
        out_shape=jax.ShapeDtypeStruct((1, 1), jnp.float32),
    )(x2, w_rows, bt, nt, bn_vals, nn_vals)
    return out[0, 0]


# speedup vs baseline: 1.1246x; 1.0048x over previous
import functools
import math

import jax
import jax.numpy as jnp
from jax import lax
from jax.experimental import pallas as pl
from jax.experimental.pallas import tpu as pltpu
from jax.experimental.pallas import tpu_sc as plsc

BACKOFF_PROB = 1e-10
CLAMP = 20.0

_NC = 2
_NS = 16
_NW = _NC * _NS


def _sc_gather_body(idx_hbm, w_hbm, b_hbm, n_hbm,
                    w_out, b_out, n_out,
                    idx_v, rows_v, b_v, n_v, sem0, sem1, sem2, b_per_w):
    wid = lax.axis_index("s") * _NC + lax.axis_index("c")
    base = wid * b_per_w
    pltpu.sync_copy(idx_hbm.at[pl.ds(base, b_per_w)], idx_v)
    cp0 = pltpu.async_copy(w_hbm.at[idx_v], rows_v, sem0)
    cp1 = pltpu.async_copy(b_hbm.at[idx_v], b_v, sem1)
    cp2 = pltpu.async_copy(n_hbm.at[idx_v], n_v, sem2)
    cp0.wait()
    cp1.wait()
    cp2.wait()
    pltpu.sync_copy(rows_v, w_out.at[pl.ds(base, b_per_w)])
    pltpu.sync_copy(b_v, b_out.at[pl.ds(base, b_per_w)])
    pltpu.sync_copy(n_v, n_out.at[pl.ds(base, b_per_w)])


def _make_sc_gather(total, d):
    b_per_w = total // _NW
    mesh = plsc.VectorSubcoreMesh(core_axis_name="c", subcore_axis_name="s")
    return pl.kernel(
        functools.partial(_sc_gather_body, b_per_w=b_per_w),
        mesh=mesh,
        out_type=[
            jax.ShapeDtypeStruct((total, d), jnp.float32),
            jax.ShapeDtypeStruct((total,), jnp.float32),
            jax.ShapeDtypeStruct((total,), jnp.float32),
        ],
        scratch_types=[
            pltpu.VMEM((b_per_w,), jnp.int32),
            pltpu.VMEM((b_per_w, d), jnp.float32),
            pltpu.VMEM((b_per_w,), jnp.float32),
            pltpu.VMEM((b_per_w,), jnp.float32),
            pltpu.SemaphoreType.DMA,
            pltpu.SemaphoreType.DMA,
            pltpu.SemaphoreType.DMA,
        ],
    )


def _tc_loss_body(x_ref, wt_ref, wn_ref, bt_ref, nt_ref, bn_ref, nn_ref,
                  out_ref, *, bn_count, k, kpad, norm_term):
    i = pl.program_id(0)

    @pl.when(i == 0)
    def _init():
        out_ref[...] = jnp.zeros_like(out_ref)

    x = x_ref[...]
    wt = wt_ref[...]
    wn = wn_ref[...]
    ts = jnp.sum(x * wt, axis=1) + bt_ref[...]
    ns = lax.dot_general(x, wn, (((1,), (1,)), ((), ())),
                         precision=lax.Precision.HIGHEST,
                         preferred_element_type=jnp.float32)
    ns = ns + bn_ref[...][None, :]
    pm = jnp.exp(jnp.minimum(ts - norm_term, CLAMP))
    pnm = jnp.exp(jnp.minimum(ns - norm_term, CLAMP))
    kpt = k * nt_ref[...]
    kpn = k * nn_ref[...][None, :]
    p_true = pm / (pm + kpt + BACKOFF_PROB)
    p_noise = kpn / (pnm + kpn + BACKOFF_PROB)
    lp = jnp.log(p_noise + BACKOFF_PROB)
    col = lax.broadcasted_iota(jnp.int32, lp.shape, 1)
    lp = jnp.where(col < k, lp, 0.0)
    loss = -(jnp.log(p_true + BACKOFF_PROB) + jnp.sum(lp, axis=1))
    out_ref[...] += jnp.sum(loss).reshape(1, 1) / bn_count


def kernel(target, input, weight, bias, noise, noise_idx):
    b, n, d = input.shape
    bn_count = b * n
    v = weight.shape[0]
    k = noise_idx.shape[0]
    norm_term = float(math.log(v))

    total = ((bn_count + k + 8 * _NW - 1) // (8 * _NW)) * (8 * _NW)
    kpad = total - bn_count

    tflat = target.reshape(-1).astype(jnp.int32)
    nidx = jnp.pad(noise_idx.astype(jnp.int32), (0, kpad - k))
    idx_all = jnp.concatenate([tflat, nidx])

    w_rows, b_rows, n_rows = _make_sc_gather(total, d)(idx_all, weight, bias, noise)

    x2 = input.reshape(bn_count, d)
    rows_per_blk = 128
    nblk = bn_count // rows_per_blk
    noise_blk = bn_count // kpad

    out = pl.pallas_call(
        functools.partial(_tc_loss_body, bn_count=bn_count, k=k, kpad=kpad,
                          norm_term=norm_term),
        grid=(nblk,),
        in_specs=[
            pl.BlockSpec((rows_per_blk, d), lambda i: (i, 0)),
            pl.BlockSpec((rows_per_blk, d), lambda i: (i, 0)),
            pl.BlockSpec((kpad, d), lambda i: (noise_blk, 0)),
            pl.BlockSpec((rows_per_blk,), lambda i: (i,)),
            pl.BlockSpec((rows_per_blk,), lambda i: (i,)),
            pl.BlockSpec((kpad,), lambda i: (noise_blk,)),
            pl.BlockSpec((kpad,), lambda i: (noise_blk,)),
        ],
        out_specs=pl.BlockSpec((1, 1), lambda i: (0, 0)),
        out_shape=jax.ShapeDtypeStruct((1, 1), jnp.float32),
    )(x2, w_rows, w_rows, b_rows, n_rows, b_rows, n_rows)
    return out[0, 0]


# speedup vs baseline: 1.2591x; 1.1196x over previous
import functools
import math

import jax
import jax.numpy as jnp
from jax import lax
from jax.experimental import pallas as pl
from jax.experimental.pallas import tpu as pltpu
from jax.experimental.pallas import tpu_sc as plsc

BACKOFF_PROB = 1e-10
CLAMP = 20.0

_NC = 2
_NS = 16
_NW = _NC * _NS


def _sc_gather_body(idx_hbm, w_hbm, b_hbm, n_hbm,
                    w_out, b_out, n_out,
                    idx_v, rows_v, b_v, n_v, sem0, sem1, sem2, b_per_w):
    wid = lax.axis_index("s") * _NC + lax.axis_index("c")
    base = wid * b_per_w
    pltpu.sync_copy(idx_hbm.at[pl.ds(base, b_per_w)], idx_v)
    cp0 = pltpu.async_copy(w_hbm.at[idx_v], rows_v, sem0)
    cp1 = pltpu.async_copy(b_hbm.at[idx_v], b_v, sem1)
    cp2 = pltpu.async_copy(n_hbm.at[idx_v], n_v, sem2)
    cp0.wait()
    cp1.wait()
    cp2.wait()
    pltpu.sync_copy(rows_v, w_out.at[pl.ds(base, b_per_w)])
    pltpu.sync_copy(b_v, b_out.at[pl.ds(base, b_per_w)])
    pltpu.sync_copy(n_v, n_out.at[pl.ds(base, b_per_w)])


def _make_sc_gather(total, d):
    b_per_w = total // _NW
    mesh = plsc.VectorSubcoreMesh(core_axis_name="c", subcore_axis_name="s")
    return pl.kernel(
        functools.partial(_sc_gather_body, b_per_w=b_per_w),
        mesh=mesh,
        out_type=[
            jax.ShapeDtypeStruct((total, d), jnp.float32),
            jax.ShapeDtypeStruct((total,), jnp.float32),
            jax.ShapeDtypeStruct((total,), jnp.float32),
        ],
        scratch_types=[
            pltpu.VMEM((b_per_w,), jnp.int32),
            pltpu.VMEM((b_per_w, d), jnp.float32),
            pltpu.VMEM((b_per_w,), jnp.float32),
            pltpu.VMEM((b_per_w,), jnp.float32),
            pltpu.SemaphoreType.DMA,
            pltpu.SemaphoreType.DMA,
            pltpu.SemaphoreType.DMA,
        ],
    )


def _tc_loss_body(x_ref, w_ref, b_ref, n_ref, out_ref,
                  *, bn_count, k, kpad, norm_term):
    x = x_ref[...]
    wt = w_ref[0:bn_count, :]
    wn = w_ref[bn_count:bn_count + kpad, :]
    ts = jnp.sum(x * wt, axis=1) + b_ref[0:bn_count]
    ns = lax.dot_general(x, wn, (((1,), (1,)), ((), ())),
                         precision=lax.Precision.HIGHEST,
                         preferred_element_type=jnp.float32)
    ns = ns + b_ref[bn_count:bn_count + kpad][None, :]
    pm = jnp.exp(jnp.minimum(ts - norm_term, CLAMP))
    pnm = jnp.exp(jnp.minimum(ns - norm_term, CLAMP))
    kpt = k * n_ref[0:bn_count]
    kpn = k * n_ref[bn_count:bn_count + kpad][None, :]
    p_true = pm / (pm + kpt + BACKOFF_PROB)
    p_noise = kpn / (pnm + kpn + BACKOFF_PROB)
    lp = jnp.log(p_noise + BACKOFF_PROB)
    col = lax.broadcasted_iota(jnp.int32, lp.shape, 1)
    lp = jnp.where(col < k, lp, 0.0)
    loss = -(jnp.log(p_true + BACKOFF_PROB) + jnp.sum(lp, axis=1))
    out_ref[...] = jnp.sum(loss).reshape(1, 1) / bn_count


def kernel(target, input, weight, bias, noise, noise_idx):
    b, n, d = input.shape
    bn_count = b * n
    v = weight.shape[0]
    k = noise_idx.shape[0]
    norm_term = float(math.log(v))

    total = ((bn_count + k + 8 * _NW - 1) // (8 * _NW)) * (8 * _NW)
    kpad = total - bn_count

    tflat = target.reshape(-1).astype(jnp.int32)
    nidx = jnp.pad(noise_idx.astype(jnp.int32), (0, kpad - k))
    idx_all = jnp.concatenate([tflat, nidx])

    w_rows, b_rows, n_rows = _make_sc_gather(total, d)(idx_all, weight, bias, noise)

    x2 = input.reshape(bn_count, d)
    out = pl.pallas_call(
        functools.partial(_tc_loss_body, bn_count=bn_count, k=k, kpad=kpad,
                          norm_term=norm_term),
        out_shape=jax.ShapeDtypeStruct((1, 1), jnp.float32),
    )(x2, w_rows, b_rows, n_rows)
    return out[0, 0]
